# Initial kernel scaffold; baseline (speedup 1.0000x reference)
import jax
import jax.numpy as jnp
from jax.experimental import pallas as pl


def kernel(x):
    raise NotImplementedError("write your pallas kernel here")


# speedup vs baseline: 26.6066x; 26.6066x over previous
import functools

import jax
import jax.numpy as jnp
from jax import lax
from jax.experimental import pallas as pl
from jax.experimental.pallas import tpu as pltpu
from jax.experimental.pallas import tpu_sc as plsc

ROWS = 2048
N = 4096
K = 128
NCHUNK = N // K
NV = K // 16

_NUM_TILES = 32
ROWS_PER_TILE = ROWS // _NUM_TILES


def _rev16(v):
  return lax.rev(v, (0,))


def _vsort(v):
  return lax.sort(v, dimension=0, is_stable=False)


def _clean_asc(c):
  m = len(c)
  if m == 1:
    return [_vsort(c[0])]
  h = m // 2
  lo = [jnp.minimum(c[i], c[i + h]) for i in range(h)]
  hi = [jnp.maximum(c[i], c[i + h]) for i in range(h)]
  return _clean_asc(lo) + _clean_asc(hi)


def _merge_asc(a, b):
  c = a + [_rev16(x) for x in reversed(b)]
  return _clean_asc(c)


def _build_run(vecs):
  s = [_vsort(v) for v in vecs]
  m01 = _merge_asc(s[0:1], s[1:2])
  m23 = _merge_asc(s[2:3], s[3:4])
  m45 = _merge_asc(s[4:5], s[5:6])
  m67 = _merge_asc(s[6:7], s[7:8])
  m0123 = _merge_asc(m01, m23)
  m4567 = _merge_asc(m45, m67)
  return _merge_asc(m0123, m4567)


def _prune_merge(acc, run):
  hi = [jnp.maximum(acc[i], _rev16(run[NV - 1 - i])) for i in range(NV)]
  return _clean_asc(hi)


def _row_topk(row_ref, base):
  def load_chunk(c):
    off = base + c * K
    return [row_ref[pl.ds(off + i * 16, 16)] for i in range(NV)]

  acc = _build_run(load_chunk(0))

  def chunk_body(c, acc):
    run = _build_run(load_chunk(c))
    return tuple(_prune_merge(list(acc), run))

  acc = lax.fori_loop(1, NCHUNK, chunk_body, tuple(acc), unroll=False)
  return list(acc)


def _sc_kernel_body(x_hbm, out_hbm, row_v, out_v, in_sem):
  wid = lax.axis_index("s") * 2 + lax.axis_index("c")
  row0 = wid * ROWS_PER_TILE

  cp0 = pltpu.make_async_copy(
      x_hbm.at[pl.ds(row0 * N, N)], row_v.at[pl.ds(0, N)], in_sem
  )
  cp0.start()

  def row_body(r, _):
    row = row0 + r
    buf = lax.rem(r, 2)
    nbuf = 1 - buf

    @pl.when(r < ROWS_PER_TILE - 1)
    def _():
      pltpu.make_async_copy(
          x_hbm.at[pl.ds((row + 1) * N, N)],
          row_v.at[pl.ds(nbuf * N, N)],
          in_sem,
      ).start()

    pltpu.make_async_copy(
        x_hbm.at[pl.ds(row * N, N)], row_v.at[pl.ds(buf * N, N)], in_sem
    ).wait()

    acc = _row_topk(row_v, buf * N)

    out_off = r * K
    for j in range(NV):
      out_v[pl.ds(out_off + j * 16, 16)] = _rev16(acc[NV - 1 - j])
    return 0

  lax.fori_loop(0, ROWS_PER_TILE, row_body, 0, unroll=False)

  pltpu.sync_copy(out_v, out_hbm.at[pl.ds(row0 * K, ROWS_PER_TILE * K)])


_mesh = plsc.VectorSubcoreMesh(core_axis_name="c", subcore_axis_name="s")

_topk_call = functools.partial(
    pl.kernel,
    out_type=jax.ShapeDtypeStruct((ROWS * K,), jnp.float32),
    mesh=_mesh,
    compiler_params=pltpu.CompilerParams(needs_layout_passes=False),
    scratch_types=[
        pltpu.VMEM((2 * N,), jnp.float32),
        pltpu.VMEM((ROWS_PER_TILE * K,), jnp.float32),
        pltpu.SemaphoreType.DMA,
    ],
)(_sc_kernel_body)


@jax.jit
def kernel(x):
  flat = x.reshape(ROWS * N)
  out = _topk_call(flat)
  return out.reshape(x.shape[0], x.shape[1], K)


# speedup vs baseline: 28.9440x; 1.0879x over previous
import functools

import jax
import jax.numpy as jnp
from jax import lax
from jax.experimental import pallas as pl
from jax.experimental.pallas import tpu as pltpu
from jax.experimental.pallas import tpu_sc as plsc

ROWS = 2048
N = 4096
K = 128
NCHUNK = N // K
NV = K // 16

_NUM_TILES = 32
ROWS_PER_TILE = ROWS // _NUM_TILES


def _rev16(v):
  return lax.rev(v, (0,))


def _vsort(v):
  return lax.sort(v, dimension=0, is_stable=False)


def _clean_asc(c):
  m = len(c)
  if m == 1:
    return [_vsort(c[0])]
  h = m // 2
  lo = [jnp.minimum(c[i], c[i + h]) for i in range(h)]
  hi = [jnp.maximum(c[i], c[i + h]) for i in range(h)]
  return _clean_asc(lo) + _clean_asc(hi)


def _build_run(vecs, sign):
  n = len(vecs)
  if n == 1:
    v = vecs[0] if sign > 0 else -vecs[0]
    return [_vsort(v)]
  h = n // 2
  left = _build_run(vecs[:h], sign)
  right = _build_run(vecs[h:], -sign)
  return _clean_asc(left + [-x for x in right])


def _prune_merge(acc, run_neg):
  hi = [jnp.maximum(acc[i], -run_neg[i]) for i in range(NV)]
  return _clean_asc(hi)


def _row_topk(row_ref, base):
  def load_chunk(c):
    off = base + c * K
    return [row_ref[pl.ds(off + i * 16, 16)] for i in range(NV)]

  acc = _build_run(load_chunk(0), 1)

  def chunk_body(c, acc):
    run_neg = _build_run(load_chunk(c), -1)
    return tuple(_prune_merge(list(acc), run_neg))

  acc = lax.fori_loop(1, NCHUNK, chunk_body, tuple(acc), unroll=False)
  return list(acc)


def _sc_kernel_body(x_hbm, out_hbm, row_v, out_v, in_sem):
  wid = lax.axis_index("s") * 2 + lax.axis_index("c")
  row0 = wid * ROWS_PER_TILE

  cp0 = pltpu.make_async_copy(
      x_hbm.at[pl.ds(row0 * N, N)], row_v.at[pl.ds(0, N)], in_sem
  )
  cp0.start()

  def row_body(r, _):
    row = row0 + r
    buf = lax.rem(r, 2)
    nbuf = 1 - buf

    @pl.when(r < ROWS_PER_TILE - 1)
    def _():
      pltpu.make_async_copy(
          x_hbm.at[pl.ds((row + 1) * N, N)],
          row_v.at[pl.ds(nbuf * N, N)],
          in_sem,
      ).start()

    pltpu.make_async_copy(
        x_hbm.at[pl.ds(row * N, N)], row_v.at[pl.ds(buf * N, N)], in_sem
    ).wait()

    acc = _row_topk(row_v, buf * N)

    out_off = r * K
    for j in range(NV):
      out_v[pl.ds(out_off + j * 16, 16)] = _rev16(acc[NV - 1 - j])
    return 0

  lax.fori_loop(0, ROWS_PER_TILE, row_body, 0, unroll=False)

  pltpu.sync_copy(out_v, out_hbm.at[pl.ds(row0 * K, ROWS_PER_TILE * K)])


_mesh = plsc.VectorSubcoreMesh(core_axis_name="c", subcore_axis_name="s")

_topk_call = functools.partial(
    pl.kernel,
    out_type=jax.ShapeDtypeStruct((ROWS * K,), jnp.float32),
    mesh=_mesh,
    compiler_params=pltpu.CompilerParams(needs_layout_passes=False),
    scratch_types=[
        pltpu.VMEM((2 * N,), jnp.float32),
        pltpu.VMEM((ROWS_PER_TILE * K,), jnp.float32),
        pltpu.SemaphoreType.DMA,
    ],
)(_sc_kernel_body)


@jax.jit
def kernel(x):
  flat = x.reshape(ROWS * N)
  out = _topk_call(flat)
  return out.reshape(x.shape[0], x.shape[1], K)
